# baseline (speedup 1.0000x reference)
import jax
import jax.numpy as jnp
from jax import lax
from jax.experimental import pallas as pl
from jax.experimental.pallas import tpu as pltpu
from jax.experimental.pallas import tpu_sc as plsc

B, T, D = 32, 500, 3000
N = B * T
NW = 32
LANES = 16
TPAD = 512
NCH = TPAD // LANES
SUB, LN = 8, 128
NBUF = 7


def _sc_body(
    x_hbm, tgt_hbm, out_hbm, tgt_v, buf, acc_v, sem0, sem1, sem2, sem3, sem4, sem5, sem6
):
    cid = lax.axis_index("c")
    sid = lax.axis_index("s")
    wid = sid * 2 + cid

    btile0 = pl.multiple_of(wid & ~(SUB - 1), SUB)
    sub = wid & (SUB - 1)

    tgt_copies = [
        pltpu.async_copy(
            tgt_hbm.at[
                pl.ds(btile0, SUB),
                pl.ds(pl.multiple_of(tt * LN + (wid & 0), LN), LN),
            ],
            tgt_v.at[tt],
            sem0,
        )
        for tt in range(TPAD // LN)
    ]
    for cp in tgt_copies:
        cp.wait()

    lane = lax.iota(jnp.int32, LANES)
    sub16 = jnp.full((LANES,), sub, jnp.int32)
    sems = [sem0, sem1, sem2, sem3, sem4, sem5, sem6]

    def chunk_targets(c):
        t16 = tgt_v[c >> 3, sub, pl.ds((c & (SUB - 1)) * LANES, LANES)]
        return jnp.where(c * LANES + lane < T, t16, 0)

    def enqueue(c, parity):
        col16 = chunk_targets(c) & ~(LN - 1)
        for j in range(LANES):
            i = jnp.minimum(c * LANES + j, T - 1)
            col0 = pl.multiple_of(col16[j], LN)
            pltpu.async_copy(
                x_hbm.at[i, pl.ds(btile0, SUB), pl.ds(col0, LN)],
                buf.at[parity, j],
                sems[parity],
            )

    def drain(parity):
        pltpu.make_async_copy(
            x_hbm.at[pl.ds(0, LANES), pl.ds(0, SUB), pl.ds(0, LN)],
            buf.at[parity],
            sems[parity],
        ).wait()

    def extract(c, parity, acc):
        t16 = chunk_targets(c)
        v16 = plsc.load_gather(buf.at[parity], [lane, sub16, t16 & (LN - 1)])
        v16 = jnp.clip(v16, -30.0, 30.0)
        v16 = jnp.where(c * LANES + lane < T, v16, 0.0)
        return acc + v16

    for p in range(NBUF):
        enqueue(jnp.int32(p), p)

    def round_body(r, acc):
        for p in range(NBUF):
            c = r * NBUF + p
            drain(p)
            acc = extract(c, p, acc)
            nc = c + NBUF

            @pl.when(nc < NCH)
            def _():
                enqueue(nc, p)

        return acc

    acc = jnp.zeros((LANES,), jnp.float32)
    nround = NCH // NBUF
    acc = lax.fori_loop(0, nround, round_body, acc)
    for p in range(NCH - nround * NBUF):
        c = jnp.int32(nround * NBUF + p)
        drain(p)
        acc = extract(c, p, acc)

    acc_v[...] = acc * (-1.0 / N)
    pltpu.sync_copy(acc_v, out_hbm.at[wid])


@jax.jit
def _chain_loss(xt, tgt):
    mesh = plsc.VectorSubcoreMesh(core_axis_name="c", subcore_axis_name="s")
    partials = pl.kernel(
        _sc_body,
        mesh=mesh,
        compiler_params=pltpu.CompilerParams(
            needs_layout_passes=False, use_tc_tiling_on_sc=True
        ),
        out_type=jax.ShapeDtypeStruct((NW, LANES), jnp.float32),
        scratch_types=[
            pltpu.VMEM((TPAD // LN, SUB, LN), jnp.int32),
            pltpu.VMEM((NBUF, LANES, SUB, LN), jnp.float32),
            pltpu.VMEM((LANES,), jnp.float32),
            pltpu.SemaphoreType.DMA,
            pltpu.SemaphoreType.DMA,
            pltpu.SemaphoreType.DMA,
            pltpu.SemaphoreType.DMA,
            pltpu.SemaphoreType.DMA,
            pltpu.SemaphoreType.DMA,
            pltpu.SemaphoreType.DMA,
        ],
    )(xt, tgt)
    return jnp.sum(partials)


def kernel(x, targets):
    xt = jnp.transpose(x, (1, 0, 2))
    return _chain_loss(xt, targets.astype(jnp.int32))


# speedup vs baseline: 1.0105x; 1.0105x over previous
import jax
import jax.numpy as jnp
from jax import lax
from jax.experimental import pallas as pl
from jax.experimental.pallas import tpu as pltpu
from jax.experimental.pallas import tpu_sc as plsc

B, T, D = 32, 500, 3000
N = B * T
NW = 32
LANES = 16
TPAD = 512
CPT = 32
NCH = TPAD // CPT
SUB, LN = 8, 128
NBUF = 3


def _sc_body(x_hbm, tgt_hbm, out_hbm, tgt_v, buf, acc_v, sem0, sem1, sem2):
    cid = lax.axis_index("c")
    sid = lax.axis_index("s")
    wid = sid * 2 + cid

    btile0 = pl.multiple_of(wid & ~(SUB - 1), SUB)
    sub = wid & (SUB - 1)

    tgt_copies = [
        pltpu.async_copy(
            tgt_hbm.at[
                pl.ds(btile0, SUB),
                pl.ds(pl.multiple_of(tt * LN + (wid & 0), LN), LN),
            ],
            tgt_v.at[tt],
            sem0,
        )
        for tt in range(TPAD // LN)
    ]
    for cp in tgt_copies:
        cp.wait()

    lane = lax.iota(jnp.int32, LANES)
    sub16 = jnp.full((LANES,), sub, jnp.int32)
    sems = [sem0, sem1, sem2]

    def chunk_targets(c):
        t16 = tgt_v[c >> 3, sub, pl.ds((c & (SUB - 1)) * LANES, LANES)]
        return jnp.where(c * LANES + lane < T, t16, 0)

    def enqueue(c, parity):
        for h in range(CPT // LANES):
            cc = c * (CPT // LANES) + h
            col16 = chunk_targets(cc) & ~(LN - 1)
            for j in range(LANES):
                i = jnp.minimum(cc * LANES + j, T - 1)
                col0 = pl.multiple_of(col16[j], LN)
                pltpu.async_copy(
                    x_hbm.at[i, pl.ds(btile0, SUB), pl.ds(col0, LN)],
                    buf.at[parity, h * LANES + j],
                    sems[parity],
                )

    def drain(parity):
        pltpu.make_async_copy(
            x_hbm.at[pl.ds(0, CPT), pl.ds(0, SUB), pl.ds(0, LN)],
            buf.at[parity],
            sems[parity],
        ).wait()

    def extract(c, parity, acc):
        for h in range(CPT // LANES):
            cc = c * (CPT // LANES) + h
            t16 = chunk_targets(cc)
            v16 = plsc.load_gather(
                buf.at[parity], [h * LANES + lane, sub16, t16 & (LN - 1)]
            )
            v16 = jnp.clip(v16, -30.0, 30.0)
            v16 = jnp.where(cc * LANES + lane < T, v16, 0.0)
            acc = acc + v16
        return acc

    for p in range(NBUF):
        enqueue(jnp.int32(p), p)

    def round_body(r, acc):
        for p in range(NBUF):
            c = r * NBUF + p
            drain(p)
            acc = extract(c, p, acc)
            nc = c + NBUF

            @pl.when(nc < NCH)
            def _():
                enqueue(nc, p)

        return acc

    acc = jnp.zeros((LANES,), jnp.float32)
    nround = NCH // NBUF
    acc = lax.fori_loop(0, nround, round_body, acc)
    for p in range(NCH - nround * NBUF):
        c = jnp.int32(nround * NBUF + p)
        drain(p)
        acc = extract(c, p, acc)

    acc_v[...] = acc * (-1.0 / N)
    pltpu.sync_copy(acc_v, out_hbm.at[wid])


@jax.jit
def _chain_loss(xt, tgt):
    mesh = plsc.VectorSubcoreMesh(core_axis_name="c", subcore_axis_name="s")
    partials = pl.kernel(
        _sc_body,
        mesh=mesh,
        compiler_params=pltpu.CompilerParams(
            needs_layout_passes=False, use_tc_tiling_on_sc=True
        ),
        out_type=jax.ShapeDtypeStruct((NW, LANES), jnp.float32),
        scratch_types=[
            pltpu.VMEM((TPAD // LN, SUB, LN), jnp.int32),
            pltpu.VMEM((NBUF, CPT, SUB, LN), jnp.float32),
            pltpu.VMEM((LANES,), jnp.float32),
            pltpu.SemaphoreType.DMA,
            pltpu.SemaphoreType.DMA,
            pltpu.SemaphoreType.DMA,
        ],
    )(xt, tgt)
    return jnp.sum(partials)


def kernel(x, targets):
    xt = jnp.transpose(x, (1, 0, 2))
    return _chain_loss(xt, targets.astype(jnp.int32))


# speedup vs baseline: 1.0119x; 1.0013x over previous
import jax
import jax.numpy as jnp
from jax import lax
from jax.experimental import pallas as pl
from jax.experimental.pallas import tpu as pltpu
from jax.experimental.pallas import tpu_sc as plsc

B, T, D = 32, 500, 3000
N = B * T
NW = 32
LANES = 16
TPAD = 512
CPT = 32
NCH = TPAD // CPT
SUB, LN = 8, 128
NBUF = 3


def _sc_body(x_hbm, tgt_hbm, out_hbm, tgt_v, buf, acc_v, sem0, sem1, sem2):
    cid = lax.axis_index("c")
    sid = lax.axis_index("s")
    wid = sid * 2 + cid

    btile0 = pl.multiple_of(wid & ~(SUB - 1), SUB)
    sub = wid & (SUB - 1)

    tgt_copies = [
        pltpu.async_copy(
            tgt_hbm.at[
                pl.ds(btile0, SUB),
                pl.ds(pl.multiple_of(tt * LN + (wid & 0), LN), LN),
            ],
            tgt_v.at[tt],
            sem0,
        )
        for tt in range(TPAD // LN)
    ]
    for cp in tgt_copies:
        cp.wait()

    lane = lax.iota(jnp.int32, LANES)
    sub16 = jnp.full((LANES,), sub, jnp.int32)
    sems = [sem0, sem1, sem2]

    def chunk_targets(c):
        t16 = tgt_v[c >> 3, sub, pl.ds((c & (SUB - 1)) * LANES, LANES)]
        return jnp.where(c * LANES + lane < T, t16, 0)

    def enqueue(c, parity):
        for h in range(CPT // LANES):
            cc = c * (CPT // LANES) + h
            col16 = chunk_targets(cc) & ~(LN - 1)
            for j in range(LANES):
                i = jnp.minimum(cc * LANES + j, T - 1)
                col0 = pl.multiple_of(col16[j], LN)
                pltpu.async_copy(
                    x_hbm.at[i, pl.ds(btile0, SUB), pl.ds(col0, LN)],
                    buf.at[parity, h * LANES + j],
                    sems[parity],
                )

    def drain(parity):
        pltpu.make_async_copy(
            x_hbm.at[pl.ds(0, CPT), pl.ds(0, SUB), pl.ds(0, LN)],
            buf.at[parity],
            sems[parity],
        ).wait()

    def extract(c, parity, acc):
        for h in range(CPT // LANES):
            cc = c * (CPT // LANES) + h
            t16 = chunk_targets(cc)
            v16 = plsc.load_gather(
                buf.at[parity], [h * LANES + lane, sub16, t16 & (LN - 1)]
            )
            v16 = jnp.clip(v16, -30.0, 30.0)
            v16 = jnp.where(cc * LANES + lane < T, v16, 0.0)
            acc = acc + v16
        return acc

    for p in range(NBUF):
        enqueue(jnp.int32(p), p)

    def round_body(r, acc):
        for p in range(NBUF):
            c = r * NBUF + p
            drain(p)
            acc = extract(c, p, acc)
            nc = c + NBUF

            @pl.when(nc < NCH)
            def _():
                enqueue(nc, p)

        return acc

    acc = jnp.zeros((LANES,), jnp.float32)
    nround = NCH // NBUF
    acc = lax.fori_loop(0, nround, round_body, acc)
    for p in range(NCH - nround * NBUF):
        c = jnp.int32(nround * NBUF + p)
        drain(p)
        acc = extract(c, p, acc)

    acc_v[...] = acc * (-1.0 / N)
    pltpu.sync_copy(acc_v, out_hbm.at[wid])


@jax.jit
def _chain_loss(xt, tgt):
    mesh = plsc.VectorSubcoreMesh(core_axis_name="c", subcore_axis_name="s")
    partials = pl.kernel(
        _sc_body,
        mesh=mesh,
        compiler_params=pltpu.CompilerParams(
            needs_layout_passes=False,
            use_tc_tiling_on_sc=True,
            disable_bounds_checks=True,
            disable_semaphore_checks=True,
        ),
        out_type=jax.ShapeDtypeStruct((NW, LANES), jnp.float32),
        scratch_types=[
            pltpu.VMEM((TPAD // LN, SUB, LN), jnp.int32),
            pltpu.VMEM((NBUF, CPT, SUB, LN), jnp.float32),
            pltpu.VMEM((LANES,), jnp.float32),
            pltpu.SemaphoreType.DMA,
            pltpu.SemaphoreType.DMA,
            pltpu.SemaphoreType.DMA,
        ],
    )(xt, tgt)
    return jnp.sum(partials)


def kernel(x, targets):
    xt = jnp.transpose(x, (1, 0, 2))
    return _chain_loss(xt, targets.astype(jnp.int32))
